# baseline (speedup 1.0000x reference)
import jax
import jax.numpy as jnp
from jax import lax
from jax.experimental import pallas as pl
from jax.experimental.pallas import tpu as pltpu
from jax.experimental.pallas import tpu_sc as plsc

H = 50
W = 50
C = 128
P = 7
NB = P * P
CPG = 64
LROWS = 33
UBASE = 25
FSZ = LROWS * W * CPG
NRG = 8
NPAD = 1024
RPW = NPAD // NRG
OSZ = NB * CPG
SPATIAL_SCALE = 0.0625


def _pool_body(feat_hbm, bnd_hbm, out_hbm, feat_v, bnd_v, out_v):
    cid = lax.axis_index("c")
    sid = lax.axis_index("s")
    wid = sid * 2 + cid
    hf = wid % 2
    cg = (wid // 2) % 2
    rg = wid // 4
    pltpu.sync_copy(feat_hbm.at[cg * 2 + hf], feat_v)
    pltpu.sync_copy(bnd_hbm.at[pl.ds(rg * RPW * 32, RPW * 32)], bnd_v)
    ybase = hf * UBASE
    ytop = 33 + hf * 17
    neg = jnp.full((16,), -jnp.inf, dtype=jnp.float32)

    def roi_body(r, carry):
        row_x = bnd_v[pl.ds(r * 32, 16)]
        row_y = bnd_v[pl.ds(r * 32 + 16, 16)]
        xsv = [row_x[j] for j in range(P)]
        xev = [row_x[8 + j] for j in range(P)]
        for b in range(NB):
            i, j = b // P, b % P
            xs = xsv[j]
            xe = xev[j]
            if j == 0:
                ysi = jnp.maximum(row_y[i], ybase)
                yei = jnp.minimum(row_y[8 + i], ytop)
            ys, ye = ysi, yei

            def y_body(y, carry):
                base = carry[4]

                @plsc.parallel_loop(xs, xe, unroll=2, carry=carry[:4])
                def xloop(x, acc):
                    a0, a1, a2, a3 = acc
                    px = base + x * CPG
                    a0 = jnp.maximum(a0, feat_v[pl.ds(px, 16)])
                    a1 = jnp.maximum(a1, feat_v[pl.ds(px + 16, 16)])
                    a2 = jnp.maximum(a2, feat_v[pl.ds(px + 32, 16)])
                    a3 = jnp.maximum(a3, feat_v[pl.ds(px + 48, 16)])
                    return (a0, a1, a2, a3)

                return tuple(xloop) + (base + W * CPG,)

            base0 = (ys - ybase) * (W * CPG)
            a0, a1, a2, a3 = lax.fori_loop(ys, ye, y_body, (neg, neg, neg, neg, base0))[:4]
            out_v[pl.ds(b * CPG, 16)] = a0
            out_v[pl.ds(b * CPG + 16, 16)] = a1
            out_v[pl.ds(b * CPG + 32, 16)] = a2
            out_v[pl.ds(b * CPG + 48, 16)] = a3
        pltpu.sync_copy(out_v, out_hbm.at[hf, cg, rg * RPW + r])
        return carry

    lax.fori_loop(0, RPW, roi_body, 0)


def kernel(input, rois):
    n = rois.shape[0]
    feat_hw = jnp.transpose(input[0], (1, 2, 0))
    lo = jnp.transpose(feat_hw[:LROWS].reshape(LROWS * W, 2, CPG), (1, 0, 2)).reshape(2, FSZ)
    up = jnp.transpose(feat_hw[UBASE:].reshape((H - UBASE) * W, 2, CPG), (1, 0, 2)).reshape(2, -1)
    up = jnp.pad(up, ((0, 0), (0, FSZ - up.shape[1])))
    feat = jnp.stack([lo[0], up[0], lo[1], up[1]])

    coords = jnp.round(rois[:, 1:] * SPATIAL_SCALE)
    x1 = coords[:, 0]
    y1 = coords[:, 1]
    x2 = coords[:, 2]
    y2 = coords[:, 3]
    roi_w = jnp.clip(x2 - x1 + 1.0, 1.0, None)
    roi_h = jnp.clip(y2 - y1 + 1.0, 1.0, None)
    bin_w = roi_w / P
    bin_h = roi_h / P
    g = jnp.arange(P, dtype=jnp.float32)
    xs = jnp.clip(jnp.floor(g[None, :] * bin_w[:, None]) + x1[:, None], 0.0, float(W)).astype(jnp.int32)
    xe = jnp.clip(jnp.ceil((g[None, :] + 1.0) * bin_w[:, None]) + x1[:, None], 0.0, float(W)).astype(jnp.int32)
    ys = jnp.clip(jnp.floor(g[None, :] * bin_h[:, None]) + y1[:, None], 0.0, float(H)).astype(jnp.int32)
    ye = jnp.clip(jnp.ceil((g[None, :] + 1.0) * bin_h[:, None]) + y1[:, None], 0.0, float(H)).astype(jnp.int32)
    pad1 = lambda a: jnp.pad(a, ((0, 0), (0, 1)))
    bnd = jnp.concatenate([pad1(xs), pad1(xe), pad1(ys), pad1(ye)], axis=1)
    bnd = jnp.pad(bnd, ((0, NPAD - n), (0, 0)))
    bnd = bnd.reshape(NPAD * 32)

    mesh = plsc.VectorSubcoreMesh(core_axis_name="c", subcore_axis_name="s")
    run = pl.kernel(
        _pool_body,
        mesh=mesh,
        out_type=jax.ShapeDtypeStruct((2, 2, NPAD, OSZ), jnp.float32),
        scratch_types=[
            pltpu.VMEM((FSZ,), jnp.float32),
            pltpu.VMEM((RPW * 32,), jnp.int32),
            pltpu.VMEM((OSZ,), jnp.float32),
        ],
    )
    out = run(feat, bnd)
    out = jnp.maximum(out[0], out[1])
    out = jnp.where(jnp.isinf(out), 0.0, out)
    out = out.reshape(2, NPAD, NB, CPG)
    out = jnp.transpose(out, (1, 0, 3, 2)).reshape(NPAD, C, NB)[:n]
    return out.reshape(n, C, P, P)


# speedup vs baseline: 1.0694x; 1.0694x over previous
import jax
import jax.numpy as jnp
from jax import lax
from jax.experimental import pallas as pl
from jax.experimental.pallas import tpu as pltpu
from jax.experimental.pallas import tpu_sc as plsc

H = 50
W = 50
C = 128
P = 7
NB = P * P
CPG = 64
LROWS = 33
UBASE = 25
FSZ = LROWS * W * CPG
NRG = 8
NPAD = 1024
RPW = NPAD // NRG
OSZ = NB * CPG
SPATIAL_SCALE = 0.0625


def _pool_body(feat_hbm, bnd_hbm, out_hbm, feat_v, bnd_v, out_v):
    cid = lax.axis_index("c")
    sid = lax.axis_index("s")
    wid = sid * 2 + cid
    hf = wid % 2
    cg = (wid // 2) % 2
    rg = wid // 4
    pltpu.sync_copy(feat_hbm.at[cg * 2 + hf], feat_v)
    pltpu.sync_copy(bnd_hbm.at[pl.ds(rg * RPW * 32, RPW * 32)], bnd_v)
    ybase = hf * UBASE
    ytop = 33 + hf * 17
    neg = jnp.full((16,), -jnp.inf, dtype=jnp.float32)

    def roi_body(r, carry):
        row_x = bnd_v[pl.ds(r * 32, 16)]
        row_y = bnd_v[pl.ds(r * 32 + 16, 16)]
        xsv = [row_x[j] for j in range(P)]
        xev = [row_x[8 + j] for j in range(P)]
        for b in range(NB):
            i, j = b // P, b % P
            xs = xsv[j]
            xe = xev[j]
            if j == 0:
                ysi = jnp.maximum(row_y[i], ybase)
                yei = jnp.minimum(row_y[8 + i], ytop)
            ys, ye = ysi, yei

            def y_body(y, carry):
                base = carry[4]

                @plsc.parallel_loop(xs, xe, unroll=2, carry=carry[:4] + (base,))
                def xloop(x, xc):
                    a0, a1, a2, a3, px = xc
                    a0 = jnp.maximum(a0, feat_v[pl.ds(px, 16)])
                    a1 = jnp.maximum(a1, feat_v[pl.ds(px + 16, 16)])
                    a2 = jnp.maximum(a2, feat_v[pl.ds(px + 32, 16)])
                    a3 = jnp.maximum(a3, feat_v[pl.ds(px + 48, 16)])
                    return (a0, a1, a2, a3, px + CPG)

                return tuple(xloop[:4]) + (base + W * CPG,)

            base0 = (ys - ybase) * (W * CPG) + xs * CPG
            a0, a1, a2, a3 = lax.fori_loop(ys, ye, y_body, (neg, neg, neg, neg, base0))[:4]
            out_v[pl.ds(b * CPG, 16)] = a0
            out_v[pl.ds(b * CPG + 16, 16)] = a1
            out_v[pl.ds(b * CPG + 32, 16)] = a2
            out_v[pl.ds(b * CPG + 48, 16)] = a3
        pltpu.sync_copy(out_v, out_hbm.at[hf, cg, rg * RPW + r])
        return carry

    lax.fori_loop(0, RPW, roi_body, 0)


def kernel(input, rois):
    n = rois.shape[0]
    feat_hw = jnp.transpose(input[0], (1, 2, 0))
    lo = jnp.transpose(feat_hw[:LROWS].reshape(LROWS * W, 2, CPG), (1, 0, 2)).reshape(2, FSZ)
    up = jnp.transpose(feat_hw[UBASE:].reshape((H - UBASE) * W, 2, CPG), (1, 0, 2)).reshape(2, -1)
    up = jnp.pad(up, ((0, 0), (0, FSZ - up.shape[1])))
    feat = jnp.stack([lo[0], up[0], lo[1], up[1]])

    coords = jnp.round(rois[:, 1:] * SPATIAL_SCALE)
    x1 = coords[:, 0]
    y1 = coords[:, 1]
    x2 = coords[:, 2]
    y2 = coords[:, 3]
    roi_w = jnp.clip(x2 - x1 + 1.0, 1.0, None)
    roi_h = jnp.clip(y2 - y1 + 1.0, 1.0, None)
    bin_w = roi_w / P
    bin_h = roi_h / P
    g = jnp.arange(P, dtype=jnp.float32)
    xs = jnp.clip(jnp.floor(g[None, :] * bin_w[:, None]) + x1[:, None], 0.0, float(W)).astype(jnp.int32)
    xe = jnp.clip(jnp.ceil((g[None, :] + 1.0) * bin_w[:, None]) + x1[:, None], 0.0, float(W)).astype(jnp.int32)
    ys = jnp.clip(jnp.floor(g[None, :] * bin_h[:, None]) + y1[:, None], 0.0, float(H)).astype(jnp.int32)
    ye = jnp.clip(jnp.ceil((g[None, :] + 1.0) * bin_h[:, None]) + y1[:, None], 0.0, float(H)).astype(jnp.int32)
    pad1 = lambda a: jnp.pad(a, ((0, 0), (0, 1)))
    bnd = jnp.concatenate([pad1(xs), pad1(xe), pad1(ys), pad1(ye)], axis=1)
    bnd = jnp.pad(bnd, ((0, NPAD - n), (0, 0)))
    bnd = bnd.reshape(NPAD * 32)

    mesh = plsc.VectorSubcoreMesh(core_axis_name="c", subcore_axis_name="s")
    run = pl.kernel(
        _pool_body,
        mesh=mesh,
        out_type=jax.ShapeDtypeStruct((2, 2, NPAD, OSZ), jnp.float32),
        scratch_types=[
            pltpu.VMEM((FSZ,), jnp.float32),
            pltpu.VMEM((RPW * 32,), jnp.int32),
            pltpu.VMEM((OSZ,), jnp.float32),
        ],
    )
    out = run(feat, bnd)
    out = jnp.maximum(out[0], out[1])
    out = jnp.where(jnp.isinf(out), 0.0, out)
    out = out.reshape(2, NPAD, NB, CPG)
    out = jnp.transpose(out, (1, 0, 3, 2)).reshape(NPAD, C, NB)[:n]
    return out.reshape(n, C, P, P)


# speedup vs baseline: 1.0715x; 1.0020x over previous
import jax
import jax.numpy as jnp
from jax import lax
from jax.experimental import pallas as pl
from jax.experimental.pallas import tpu as pltpu
from jax.experimental.pallas import tpu_sc as plsc

H = 50
W = 50
C = 128
P = 7
NB = P * P
CPG = 64
LROWS = 33
UBASE = 25
FSZ = LROWS * W * CPG
NRG = 8
NPAD = 1024
RPW = NPAD // NRG
OSZ = NB * CPG
SPATIAL_SCALE = 0.0625


def _pool_body(feat_hbm, bnd_hbm, out_hbm, feat_v, bnd_v, out_v, sem):
    cid = lax.axis_index("c")
    sid = lax.axis_index("s")
    wid = sid * 2 + cid
    hf = wid % 2
    cg = (wid // 2) % 2
    rg = wid // 4
    pltpu.sync_copy(feat_hbm.at[cg * 2 + hf], feat_v)
    pltpu.sync_copy(bnd_hbm.at[pl.ds(rg * RPW * 32, RPW * 32)], bnd_v)
    ybase = hf * UBASE
    ytop = 33 + hf * 17
    neg = jnp.full((16,), -jnp.inf, dtype=jnp.float32)

    def roi_body(r, carry):
        rb = r % 2
        row_x = bnd_v[pl.ds(r * 32, 16)]
        row_y = bnd_v[pl.ds(r * 32 + 16, 16)]
        @pl.when(r >= 2)
        def _():
            pltpu.make_async_copy(out_v.at[0], out_hbm.at[0, 0, 0], sem).wait()
        xsv = [row_x[j] for j in range(P)]
        xev = [row_x[8 + j] for j in range(P)]
        for b in range(NB):
            i, j = b // P, b % P
            xs = xsv[j]
            xe = xev[j]
            if j == 0:
                ysi = jnp.maximum(row_y[i], ybase)
                yei = jnp.minimum(row_y[8 + i], ytop)
            ys, ye = ysi, yei

            def y_body(y, carry):
                base = carry[4]

                def x_body(x, xc):
                    a0, a1, a2, a3, px = xc
                    a0 = jnp.maximum(a0, feat_v[pl.ds(px, 16)])
                    a1 = jnp.maximum(a1, feat_v[pl.ds(px + 16, 16)])
                    a2 = jnp.maximum(a2, feat_v[pl.ds(px + 32, 16)])
                    a3 = jnp.maximum(a3, feat_v[pl.ds(px + 48, 16)])
                    return (a0, a1, a2, a3, px + CPG)

                acc = lax.fori_loop(xs, xe, x_body, carry[:4] + (base,))[:4]
                return acc + (base + W * CPG,)

            base0 = (ys - ybase) * (W * CPG) + xs * CPG
            a0, a1, a2, a3 = lax.fori_loop(ys, ye, y_body, (neg, neg, neg, neg, base0))[:4]
            out_v[rb, pl.ds(b * CPG, 16)] = a0
            out_v[rb, pl.ds(b * CPG + 16, 16)] = a1
            out_v[rb, pl.ds(b * CPG + 32, 16)] = a2
            out_v[rb, pl.ds(b * CPG + 48, 16)] = a3
        pltpu.async_copy(out_v.at[rb], out_hbm.at[hf, cg, rg * RPW + r], sem)
        return carry

    lax.fori_loop(0, RPW, roi_body, 0)
    pltpu.make_async_copy(out_v.at[0], out_hbm.at[0, 0, 0], sem).wait()
    pltpu.make_async_copy(out_v.at[0], out_hbm.at[0, 0, 0], sem).wait()


def kernel(input, rois):
    n = rois.shape[0]
    feat_hw = jnp.transpose(input[0], (1, 2, 0))
    lo = jnp.transpose(feat_hw[:LROWS].reshape(LROWS * W, 2, CPG), (1, 0, 2)).reshape(2, FSZ)
    up = jnp.transpose(feat_hw[UBASE:].reshape((H - UBASE) * W, 2, CPG), (1, 0, 2)).reshape(2, -1)
    up = jnp.pad(up, ((0, 0), (0, FSZ - up.shape[1])))
    feat = jnp.stack([lo[0], up[0], lo[1], up[1]])

    coords = jnp.round(rois[:, 1:] * SPATIAL_SCALE)
    x1 = coords[:, 0]
    y1 = coords[:, 1]
    x2 = coords[:, 2]
    y2 = coords[:, 3]
    roi_w = jnp.clip(x2 - x1 + 1.0, 1.0, None)
    roi_h = jnp.clip(y2 - y1 + 1.0, 1.0, None)
    bin_w = roi_w / P
    bin_h = roi_h / P
    g = jnp.arange(P, dtype=jnp.float32)
    xs = jnp.clip(jnp.floor(g[None, :] * bin_w[:, None]) + x1[:, None], 0.0, float(W)).astype(jnp.int32)
    xe = jnp.clip(jnp.ceil((g[None, :] + 1.0) * bin_w[:, None]) + x1[:, None], 0.0, float(W)).astype(jnp.int32)
    ys = jnp.clip(jnp.floor(g[None, :] * bin_h[:, None]) + y1[:, None], 0.0, float(H)).astype(jnp.int32)
    ye = jnp.clip(jnp.ceil((g[None, :] + 1.0) * bin_h[:, None]) + y1[:, None], 0.0, float(H)).astype(jnp.int32)
    pad1 = lambda a: jnp.pad(a, ((0, 0), (0, 1)))
    bnd = jnp.concatenate([pad1(xs), pad1(xe), pad1(ys), pad1(ye)], axis=1)
    bnd = jnp.pad(bnd, ((0, NPAD - n), (0, 0)))
    bnd = bnd.reshape(NPAD * 32)

    mesh = plsc.VectorSubcoreMesh(core_axis_name="c", subcore_axis_name="s")
    run = pl.kernel(
        _pool_body,
        mesh=mesh,
        out_type=jax.ShapeDtypeStruct((2, 2, NPAD, OSZ), jnp.float32),
        scratch_types=[
            pltpu.VMEM((FSZ,), jnp.float32),
            pltpu.VMEM((RPW * 32,), jnp.int32),
            pltpu.VMEM((2, OSZ), jnp.float32),
            pltpu.SemaphoreType.DMA,
        ],
    )
    out = run(feat, bnd)
    out = jnp.maximum(out[0], out[1])
    out = jnp.where(jnp.isinf(out), 0.0, out)
    out = out.reshape(2, NPAD, NB, CPG)
    out = jnp.transpose(out, (1, 0, 3, 2)).reshape(NPAD, C, NB)[:n]
    return out.reshape(n, C, P, P)
